# Initial kernel scaffold; baseline (speedup 1.0000x reference)
import jax
import jax.numpy as jnp
from jax.experimental import pallas as pl


def kernel(hidden_states, router_w, w_gate, w_up, w_down):
    raise NotImplementedError("write your pallas kernel here")


# speedup vs baseline: 4.9801x; 4.9801x over previous
import functools

import jax
import jax.numpy as jnp
from jax.experimental import pallas as pl
from jax.experimental.pallas import tpu as pltpu

H = 2048


def _i0():
    return jnp.int32(0)
F = 1408
E = 8
K = 2


def _router_kernel(x_ref, rw_ref, dw_ref):
    x = x_ref[...].astype(jnp.float32)
    logits = jax.lax.dot_general(
        x, rw_ref[...], (((1,), (1,)), ((), ())),
        preferred_element_type=jnp.float32,
        precision=jax.lax.Precision.HIGHEST)
    T = logits.shape[0]
    iota = jax.lax.broadcasted_iota(jnp.int32, (T, E), 1)
    m0 = jnp.max(logits, axis=1, keepdims=True)
    i0 = jnp.min(jnp.where(logits == m0, iota, E), axis=1, keepdims=True)
    mask0 = iota == i0
    lm = jnp.where(mask0, -jnp.inf, logits)
    m1 = jnp.max(lm, axis=1, keepdims=True)
    i1 = jnp.min(jnp.where(lm == m1, iota, E), axis=1, keepdims=True)
    mask1 = iota == i1
    s = jnp.exp(m1 - m0)
    w0 = 1.0 / (1.0 + s)
    w1 = 1.0 - w0
    dw_ref[...] = jnp.where(mask0, w0, 0.0) + jnp.where(mask1, w1, 0.0)


def _expert_kernel(x_ref, wg_ref, wu_ref, wd_ref, dw_ref, out_ref, acc_ref):
    e = pl.program_id(0)
    i = pl.program_id(1)
    tm = x_ref.shape[0]
    x = x_ref[...]
    g = jax.lax.dot_general(
        x, wg_ref[0], (((1,), (1,)), ((), ())),
        preferred_element_type=jnp.float32)
    u = jax.lax.dot_general(
        x, wu_ref[0], (((1,), (1,)), ((), ())),
        preferred_element_type=jnp.float32)
    g = g.astype(jnp.bfloat16).astype(jnp.float32)
    u = u.astype(jnp.bfloat16).astype(jnp.float32)
    h = (g * jax.nn.sigmoid(g) * u).astype(jnp.bfloat16)
    o = jax.lax.dot_general(
        h, wd_ref[0], (((1,), (1,)), ((), ())),
        preferred_element_type=jnp.float32)
    iota = jax.lax.broadcasted_iota(jnp.int32, (tm, E), 1)
    wcol = jnp.sum(jnp.where(iota == e, dw_ref[...], 0.0), axis=1,
                   keepdims=True).astype(jnp.bfloat16).astype(jnp.float32)
    contrib = o * wcol
    sl = pl.ds(i * tm, tm)

    @pl.when(e == 0)
    def _():
        acc_ref[sl, :] = contrib

    @pl.when(e != 0)
    def _():
        acc_ref[sl, :] += contrib

    @pl.when(e == E - 1)
    def _():
        out_ref[...] = acc_ref[sl, :].astype(jnp.bfloat16)


@jax.jit
def kernel(hidden_states, router_w, w_gate, w_up, w_down):
    shape = hidden_states.shape
    x = hidden_states.reshape(-1, H)
    T = x.shape[0]
    rw = router_w.astype(jnp.float32)

    dense_w = pl.pallas_call(
        _router_kernel,
        out_shape=jax.ShapeDtypeStruct((T, E), jnp.float32),
        in_specs=[
            pl.BlockSpec((T, H), lambda: (_i0(), _i0())),
            pl.BlockSpec((E, H), lambda: (_i0(), _i0())),
        ],
        out_specs=pl.BlockSpec((T, E), lambda: (_i0(), _i0())),
    )(x, rw)

    NI = 8
    TM = T // NI
    out = pl.pallas_call(
        _expert_kernel,
        grid=(E, NI),
        out_shape=jax.ShapeDtypeStruct((T, H), jnp.bfloat16),
        in_specs=[
            pl.BlockSpec((TM, H), lambda e, i: (i, _i0())),
            pl.BlockSpec((1, F, H), lambda e, i: (e, _i0(), _i0())),
            pl.BlockSpec((1, F, H), lambda e, i: (e, _i0(), _i0())),
            pl.BlockSpec((1, H, F), lambda e, i: (e, _i0(), _i0())),
            pl.BlockSpec((TM, E), lambda e, i: (i, _i0())),
        ],
        out_specs=pl.BlockSpec(
            (TM, H),
            lambda e, i: (jax.lax.mul(i, jax.lax.div(e, jnp.int32(E - 1))), _i0())),
        scratch_shapes=[pltpu.VMEM((T, H), jnp.float32)],
        compiler_params=pltpu.CompilerParams(
            dimension_semantics=("arbitrary", "arbitrary")),
    )(x, w_gate, w_up, w_down, dense_w)

    return out.reshape(shape)


# speedup vs baseline: 8.8238x; 1.7718x over previous
import functools

import jax
import jax.numpy as jnp
from jax import lax
from jax.experimental import pallas as pl
from jax.experimental.pallas import tpu as pltpu
from jax.experimental.pallas import tpu_sc as plsc

H = 2048
F = 1408
E = 8
T = 2048
TM = 256
NP = (T * 2 + E * TM) // TM
TBL = 64
NC = 2
NS = 16
NW = NC * NS
CH = T // NW
HW = H // 2


def _i0():
    return jnp.int32(0)


def _pack(f32_full):
    bits = lax.bitcast_convert_type(f32_full, jnp.int32)
    lo = lax.shift_right_logical(bits[:, :HW], jnp.int32(16))
    hi = lax.bitwise_and(bits[:, HW:], jnp.int32(-65536))
    return lax.bitwise_or(lo, hi)


def _unpack(words):
    lo = lax.bitcast_convert_type(lax.shift_left(words, jnp.int32(16)), jnp.float32)
    hi = lax.bitcast_convert_type(
        lax.bitwise_and(words, jnp.int32(-65536)), jnp.float32)
    return jnp.concatenate([lo, hi], axis=1)


def _plan_kernel(x_ref, rw_ref, p0_ref, p1_ref, w0_ref, w1_ref, tbl_ref,
                 xp_ref):
    xf = x_ref[...].astype(jnp.float32)
    xp_ref[...] = _pack(xf)
    logits = lax.dot_general(
        xf, rw_ref[...], (((1,), (1,)), ((), ())),
        preferred_element_type=jnp.float32,
        precision=lax.Precision.HIGHEST)
    iota_e = lax.broadcasted_iota(jnp.int32, (T, E), 1)
    m0 = jnp.max(logits, axis=1, keepdims=True)
    i0 = jnp.min(jnp.where(logits == m0, iota_e, E), axis=1, keepdims=True)
    oh0 = iota_e == i0
    lm = jnp.where(oh0, -jnp.inf, logits)
    m1 = jnp.max(lm, axis=1, keepdims=True)
    i1 = jnp.min(jnp.where(lm == m1, iota_e, E), axis=1, keepdims=True)
    oh1 = iota_e == i1
    s = jnp.exp(m1 - m0)
    w0 = 1.0 / (1.0 + s)
    w0_ref[...] = w0
    w1_ref[...] = 1.0 - w0

    m = (oh0.astype(jnp.bfloat16) + oh1.astype(jnp.bfloat16))
    ra = lax.broadcasted_iota(jnp.int32, (T, T), 0)
    ca = lax.broadcasted_iota(jnp.int32, (T, T), 1)
    tri = (ca <= ra).astype(jnp.bfloat16)
    c = lax.dot_general(
        tri, m, (((1,), (0,)), ((), ())),
        preferred_element_type=jnp.float32)
    cnt = c[T - 1:T, :]
    pcnt = jnp.floor((cnt + (TM - 1.0)) * (1.0 / TM)) * TM
    lane8 = lax.broadcasted_iota(jnp.int32, (1, E), 1)
    incl = pcnt
    for sh in (1, 2, 4):
        incl = incl + jnp.where(lane8 >= sh, jnp.roll(incl, sh, axis=1), 0.0)
    po = incl - pcnt
    total = jnp.sum(pcnt, axis=1, keepdims=True)
    pos = po + c - 1.0
    p0 = jnp.sum(jnp.where(oh0, pos, 0.0), axis=1, keepdims=True)
    p1 = jnp.sum(jnp.where(oh1, pos, 0.0), axis=1, keepdims=True)
    p0_ref[...] = p0.astype(jnp.int32)
    p1_ref[...] = p1.astype(jnp.int32)

    lane = lax.broadcasted_iota(jnp.int32, (1, E), 1)
    sio = lax.broadcasted_iota(jnp.int32, (1, TBL), 1).astype(jnp.float32)
    s_start = jnp.minimum(sio * TM, total - TM)
    te = jnp.zeros((1, TBL), jnp.float32)
    for e in range(1, E):
        po_e = jnp.sum(jnp.where(lane == e, po, 0.0), axis=1, keepdims=True)
        te = te + jnp.minimum(jnp.maximum(s_start - po_e + 1.0, 0.0), 1.0)
    n_tiles = total * (1.0 / TM)
    col = lax.broadcasted_iota(jnp.int32, (1, TBL), 1)
    tbl_ref[...] = jnp.where(col == NP, n_tiles, te).astype(jnp.int32)


def _gemm_kernel(tbl_ref, xs_ref, wg_ref, wu_ref, wd_ref, o_ref):
    step = pl.program_id(0)

    @pl.when(step < tbl_ref[NP])
    def _():
        xb = _unpack(xs_ref[...]).astype(jnp.bfloat16)
        g = lax.dot_general(
            xb, wg_ref[0], (((1,), (1,)), ((), ())),
            preferred_element_type=jnp.float32)
        u = lax.dot_general(
            xb, wu_ref[0], (((1,), (1,)), ((), ())),
            preferred_element_type=jnp.float32)
        g = g.astype(jnp.bfloat16).astype(jnp.float32)
        u = u.astype(jnp.bfloat16).astype(jnp.float32)
        h = (g * jax.nn.sigmoid(g) * u).astype(jnp.bfloat16)
        o = lax.dot_general(
            h, wd_ref[0], (((1,), (1,)), ((), ())),
            preferred_element_type=jnp.float32)
        o_ref[...] = _pack(o.astype(jnp.bfloat16).astype(jnp.float32))


def _combine_kernel(o0_ref, o1_ref, w0_ref, w1_ref, out_ref):
    w0 = w0_ref[...].astype(jnp.bfloat16).astype(jnp.float32)
    w1 = w1_ref[...].astype(jnp.bfloat16).astype(jnp.float32)
    out_ref[...] = (w0 * _unpack(o0_ref[...]) +
                    w1 * _unpack(o1_ref[...])).astype(jnp.bfloat16)


@functools.lru_cache(maxsize=1)
def _sc_kernels():
    mesh = plsc.VectorSubcoreMesh(core_axis_name="c", subcore_axis_name="s")

    @functools.partial(
        pl.kernel, mesh=mesh,
        out_type=jax.ShapeDtypeStruct((NP * TM, HW), jnp.int32),
        scratch_types=[
            pltpu.VMEM((CH,), jnp.int32),
            pltpu.VMEM((CH,), jnp.int32),
            pltpu.VMEM((CH, HW), jnp.int32),
            pltpu.SemaphoreType.DMA,
        ],
    )
    def _sc_dispatch(x_hbm, p0_hbm, p1_hbm, xs_hbm, i0_v, i1_v, rows_v, sem):
        wid = lax.axis_index("s") * NC + lax.axis_index("c")
        base = wid * CH
        pltpu.sync_copy(x_hbm.at[pl.ds(base, CH)], rows_v)
        pltpu.sync_copy(p0_hbm.at[pl.ds(base, CH)], i0_v)
        pltpu.sync_copy(p1_hbm.at[pl.ds(base, CH)], i1_v)
        c0 = pltpu.async_copy(rows_v, xs_hbm.at[i0_v], sem)
        c1 = pltpu.async_copy(rows_v, xs_hbm.at[i1_v], sem)
        c0.wait()
        c1.wait()

    @functools.partial(
        pl.kernel, mesh=mesh,
        out_type=[jax.ShapeDtypeStruct((T, HW), jnp.int32),
                  jax.ShapeDtypeStruct((T, HW), jnp.int32)],
        scratch_types=[
            pltpu.VMEM((CH,), jnp.int32),
            pltpu.VMEM((CH,), jnp.int32),
            pltpu.VMEM((CH, HW), jnp.int32),
            pltpu.SemaphoreType.DMA,
        ],
    )
    def _sc_collect(o_hbm, p0_hbm, p1_hbm, o0_hbm, o1_hbm, i0_v, i1_v,
                    rows_v, sem):
        wid = lax.axis_index("s") * NC + lax.axis_index("c")
        base = wid * CH
        pltpu.sync_copy(p0_hbm.at[pl.ds(base, CH)], i0_v)
        pltpu.sync_copy(p1_hbm.at[pl.ds(base, CH)], i1_v)
        pltpu.async_copy(o_hbm.at[i0_v], rows_v, sem).wait()
        pltpu.sync_copy(rows_v, o0_hbm.at[pl.ds(base, CH)])
        pltpu.async_copy(o_hbm.at[i1_v], rows_v, sem).wait()
        pltpu.sync_copy(rows_v, o1_hbm.at[pl.ds(base, CH)])

    return _sc_dispatch, _sc_collect


@jax.jit
def kernel(hidden_states, router_w, w_gate, w_up, w_down):
    shape = hidden_states.shape
    x = hidden_states.reshape(T, H)
    rw = router_w.astype(jnp.float32)

    p0, p1, w0, w1, tbl, xp = pl.pallas_call(
        _plan_kernel,
        out_shape=[
            jax.ShapeDtypeStruct((T, 1), jnp.int32),
            jax.ShapeDtypeStruct((T, 1), jnp.int32),
            jax.ShapeDtypeStruct((T, 1), jnp.float32),
            jax.ShapeDtypeStruct((T, 1), jnp.float32),
            jax.ShapeDtypeStruct((1, TBL), jnp.int32),
            jax.ShapeDtypeStruct((T, HW), jnp.int32),
        ],
        in_specs=[
            pl.BlockSpec((T, H), lambda: (_i0(), _i0())),
            pl.BlockSpec((E, H), lambda: (_i0(), _i0())),
        ],
        out_specs=[
            pl.BlockSpec((T, 1), lambda: (_i0(), _i0())),
            pl.BlockSpec((T, 1), lambda: (_i0(), _i0())),
            pl.BlockSpec((T, 1), lambda: (_i0(), _i0())),
            pl.BlockSpec((T, 1), lambda: (_i0(), _i0())),
            pl.BlockSpec((1, TBL), lambda: (_i0(), _i0())),
            pl.BlockSpec((T, HW), lambda: (_i0(), _i0())),
        ],
    )(x, rw)

    p0f = p0.reshape(T)
    p1f = p1.reshape(T)
    tblf = tbl.reshape(TBL)

    sc_dispatch, sc_collect = _sc_kernels()
    xs = sc_dispatch(xp, p0f, p1f)

    o = pl.pallas_call(
        _gemm_kernel,
        grid_spec=pltpu.PrefetchScalarGridSpec(
            num_scalar_prefetch=1,
            grid=(NP,),
            in_specs=[
                pl.BlockSpec((TM, HW), lambda s, tbl: (s, _i0())),
                pl.BlockSpec((1, F, H), lambda s, tbl: (tbl[s], _i0(), _i0())),
                pl.BlockSpec((1, F, H), lambda s, tbl: (tbl[s], _i0(), _i0())),
                pl.BlockSpec((1, H, F), lambda s, tbl: (tbl[s], _i0(), _i0())),
            ],
            out_specs=pl.BlockSpec((TM, HW), lambda s, tbl: (s, _i0())),
        ),
        out_shape=jax.ShapeDtypeStruct((NP * TM, HW), jnp.int32),
    )(tblf, xs, w_gate, w_up, w_down)

    o0, o1 = sc_collect(o, p0f, p1f)

    out = pl.pallas_call(
        _combine_kernel,
        out_shape=jax.ShapeDtypeStruct((T, H), jnp.bfloat16),
        in_specs=[
            pl.BlockSpec((T, HW), lambda: (_i0(), _i0())),
            pl.BlockSpec((T, HW), lambda: (_i0(), _i0())),
            pl.BlockSpec((T, 1), lambda: (_i0(), _i0())),
            pl.BlockSpec((T, 1), lambda: (_i0(), _i0())),
        ],
        out_specs=pl.BlockSpec((T, H), lambda: (_i0(), _i0())),
    )(o0, o1, w0, w1)

    return out.reshape(shape)
